# Initial kernel scaffold; baseline (speedup 1.0000x reference)
import jax
import jax.numpy as jnp
from jax.experimental import pallas as pl


def kernel(per_atom_property, atomic_subsystem_indices):
    raise NotImplementedError("write your pallas kernel here")


# speedup vs baseline: 28.6519x; 28.6519x over previous
import functools

import jax
import jax.numpy as jnp
from jax import lax
from jax.experimental import pallas as pl
from jax.experimental.pallas import tpu as pltpu
from jax.experimental.pallas import tpu_sc as plsc

N_ATOMS_EXPECTED = 6400000
NUM_MOL = 100000
NW = 32
MPW = 3136
MPW_LAST = NUM_MOL - (NW - 1) * MPW
ACC_PAD = MPW
W = 16384
WE = W - 16


def _sc_body(n_atoms, vals_hbm, idx_hbm, bnd_hbm, out_hbm, vbuf, ibuf, acc,
             sbnd):
    c = lax.axis_index("c")
    s = lax.axis_index("s")
    w = s * 2 + c

    pltpu.sync_copy(bnd_hbm, sbnd)
    vb = sbnd[pl.ds(w, 16)]
    a0 = vb[0]
    a1 = vb[1]
    mw0 = w * MPW

    def zbody(i, _):
        acc[pl.ds(i * 16, 16)] = jnp.zeros((16,), jnp.float32)
        return 0

    lax.fori_loop(0, ACC_PAD // 16, zbody, 0)

    A0 = (a0 // 16) * 16
    nwin = (a1 - A0 + WE - 1) // WE
    iota = lax.iota(jnp.int32, 16)

    def wbody(k, carry):
        p0 = A0 + k * WE
        st = jnp.minimum(p0, n_atoms - W)
        st = pl.multiple_of(st, 16)
        pltpu.sync_copy(vals_hbm.at[pl.ds(st, W)], vbuf)
        pltpu.sync_copy(idx_hbm.at[pl.ds(st, W)], ibuf.at[pl.ds(0, W)])
        base = p0 - st
        stop = jnp.minimum(p0 + WE, a1)
        nblk = (stop - p0 + 15) // 16

        def bbody(j, cin):
            o = base + j * 16
            pos0 = p0 + j * 16
            v_raw = vbuf[pl.ds(o, 16)]
            ic = ibuf[pl.ds(o, 16)]
            inx = ibuf[pl.ds(o + 1, 16)]
            posv = pos0 + iota
            valid = (posv >= a0) & (posv < a1)
            v = jnp.where(valid, v_raw, 0.0)
            cum = plsc.cumsum(v)
            S = cum + cin
            mb = valid & ((ic != inx) | (posv == a1 - 1))
            msub = mb & (posv + 1 < a1)
            li = jnp.where(mb, ic - mw0, 0)
            ln = jnp.where(msub, inx - mw0, 0)
            plsc.addupdate_scatter(acc, [li], S, mask=mb)
            plsc.addupdate_scatter(acc, [ln], -S, mask=msub)
            return cin + cum[15]

        return lax.fori_loop(0, nblk, bbody, carry)

    lax.fori_loop(0, nwin, wbody, jnp.float32(0.0))

    @pl.when(w < NW - 1)
    def _():
        pltpu.sync_copy(acc.at[pl.ds(0, MPW)], out_hbm.at[pl.ds(mw0, MPW)])

    @pl.when(w == NW - 1)
    def _():
        pltpu.sync_copy(acc.at[pl.ds(0, MPW_LAST)],
                        out_hbm.at[pl.ds(mw0, MPW_LAST)])


@jax.jit
def kernel(per_atom_property, atomic_subsystem_indices):
    n_atoms = per_atom_property.shape[0]
    idx32 = atomic_subsystem_indices.astype(jnp.int32)
    targets = jnp.arange(NW, dtype=jnp.int32) * MPW
    bnd = jnp.searchsorted(idx32, targets, side="left").astype(jnp.int32)
    bnd = jnp.concatenate(
        [bnd, jnp.full((16,), n_atoms, dtype=jnp.int32)])

    mesh = plsc.VectorSubcoreMesh(core_axis_name="c", subcore_axis_name="s")
    fn = pl.kernel(
        functools.partial(_sc_body, n_atoms),
        mesh=mesh,
        compiler_params=pltpu.CompilerParams(needs_layout_passes=False),
        out_type=jax.ShapeDtypeStruct((NUM_MOL,), jnp.float32),
        scratch_types=[
            pltpu.VMEM((W,), jnp.float32),
            pltpu.VMEM((W + 16,), jnp.int32),
            pltpu.VMEM((ACC_PAD,), jnp.float32),
            pltpu.VMEM((48,), jnp.int32),
        ],
    )
    return fn(per_atom_property, idx32, bnd)


# speedup vs baseline: 29.1088x; 1.0159x over previous
import functools

import jax
import jax.numpy as jnp
from jax import lax
from jax.experimental import pallas as pl
from jax.experimental.pallas import tpu as pltpu
from jax.experimental.pallas import tpu_sc as plsc

N_ATOMS_EXPECTED = 6400000
NUM_MOL = 100000
NW = 32
MPW = 3136
MPW_LAST = NUM_MOL - (NW - 1) * MPW
ACC_PAD = MPW
W = 16384
WE = W - 16


def _sc_body(n_atoms, vals_hbm, idx_hbm, bnd_hbm, out_hbm, vbuf, ibuf, acc,
             sbnd):
    c = lax.axis_index("c")
    s = lax.axis_index("s")
    w = s * 2 + c

    pltpu.sync_copy(bnd_hbm, sbnd)
    vb = sbnd[pl.ds(w, 16)]
    a0 = vb[0]
    a1 = vb[1]
    mw0 = w * MPW

    def zbody(i, _):
        acc[pl.ds(i * 16, 16)] = jnp.zeros((16,), jnp.float32)
        return 0

    lax.fori_loop(0, ACC_PAD // 16, zbody, 0)

    A0 = (a0 // 16) * 16
    nwin = (a1 - A0 + WE - 1) // WE
    iota = lax.iota(jnp.int32, 16)

    def wbody(k, carry_vec):
        p0 = A0 + k * WE
        st = jnp.minimum(p0, n_atoms - W)
        st = pl.multiple_of(st, 16)
        pltpu.sync_copy(vals_hbm.at[pl.ds(st, W)], vbuf.at[pl.ds(0, W)])
        pltpu.sync_copy(idx_hbm.at[pl.ds(st, W)], ibuf.at[pl.ds(0, W)])
        base = p0 - st

        def bbody(j, cin):
            o = base + j * 16
            pos0 = p0 + j * 16
            v_raw = vbuf[pl.ds(o, 16)]
            ic = ibuf[pl.ds(o, 16)]
            inx = ibuf[pl.ds(o + 1, 16)]
            posv = pos0 + iota
            valid = (posv >= a0) & (posv < a1)
            v = jnp.where(valid, v_raw, 0.0)
            cum = plsc.cumsum(v)
            S = cum + cin
            mb = valid & ((ic != inx) | (posv == a1 - 1))
            msub = mb & (posv < a1 - 1)
            li = jnp.where(mb, ic - mw0, 0)
            ln = jnp.where(msub, inx - mw0, 0)
            plsc.addupdate_scatter(acc, [li], S, mask=mb)
            plsc.addupdate_scatter(acc, [ln], -S, mask=msub)
            return cin + cum[15]

        return lax.fori_loop(0, WE // 16, bbody, carry_vec, unroll=8)

    lax.fori_loop(0, nwin, wbody, jnp.float32(0.0))

    @pl.when(w < NW - 1)
    def _():
        pltpu.sync_copy(acc.at[pl.ds(0, MPW)], out_hbm.at[pl.ds(mw0, MPW)])

    @pl.when(w == NW - 1)
    def _():
        pltpu.sync_copy(acc.at[pl.ds(0, MPW_LAST)],
                        out_hbm.at[pl.ds(mw0, MPW_LAST)])


@jax.jit
def kernel(per_atom_property, atomic_subsystem_indices):
    n_atoms = per_atom_property.shape[0]
    idx32 = atomic_subsystem_indices.astype(jnp.int32)
    targets = jnp.arange(NW, dtype=jnp.int32) * MPW
    bnd = jnp.searchsorted(idx32, targets, side="left").astype(jnp.int32)
    bnd = jnp.concatenate(
        [bnd, jnp.full((16,), n_atoms, dtype=jnp.int32)])

    mesh = plsc.VectorSubcoreMesh(core_axis_name="c", subcore_axis_name="s")
    fn = pl.kernel(
        functools.partial(_sc_body, n_atoms),
        mesh=mesh,
        compiler_params=pltpu.CompilerParams(needs_layout_passes=False),
        out_type=jax.ShapeDtypeStruct((NUM_MOL,), jnp.float32),
        scratch_types=[
            pltpu.VMEM((2 * W,), jnp.float32),
            pltpu.VMEM((2 * W + 16,), jnp.int32),
            pltpu.VMEM((ACC_PAD,), jnp.float32),
            pltpu.VMEM((48,), jnp.int32),
        ],
    )
    return fn(per_atom_property, idx32, bnd)


# speedup vs baseline: 31.0176x; 1.0656x over previous
import functools

import jax
import jax.numpy as jnp
from jax import lax
from jax.experimental import pallas as pl
from jax.experimental.pallas import tpu as pltpu
from jax.experimental.pallas import tpu_sc as plsc

N_ATOMS_EXPECTED = 6400000
NUM_MOL = 100000
NW = 32
MPW = 3136
MPW_LAST = NUM_MOL - (NW - 1) * MPW
ACC_PAD = MPW
W = 16384
WE = W - 16


def _sc_body(n_atoms, vals_hbm, idx_hbm, bnd_hbm, out_hbm, vbuf, ibuf, acc,
             sbnd):
    c = lax.axis_index("c")
    s = lax.axis_index("s")
    w = s * 2 + c

    pltpu.sync_copy(bnd_hbm, sbnd)
    vb = sbnd[pl.ds(w, 16)]
    a0 = vb[0]
    a1 = vb[1]
    mw0 = w * MPW

    def zbody(i, _):
        acc[pl.ds(i * 16, 16)] = jnp.zeros((16,), jnp.float32)
        return 0

    lax.fori_loop(0, ACC_PAD // 16, zbody, 0)

    A0 = (a0 // 16) * 16
    iota = lax.iota(jnp.int32, 16)
    nfull = jnp.maximum(a1 - 1 - A0, 0) // WE

    def lean_w(k, carry):
        p0 = A0 + k * WE
        p0 = pl.multiple_of(p0, 16)
        pltpu.sync_copy(vals_hbm.at[pl.ds(p0, W)], vbuf.at[pl.ds(0, W)])
        pltpu.sync_copy(idx_hbm.at[pl.ds(p0, W)], ibuf.at[pl.ds(0, W)])

        @pl.when(k == 0)
        def _():
            pad = a0 - A0
            x = vbuf[pl.ds(0, 16)]
            vbuf[pl.ds(0, 16)] = jnp.where(iota < pad, 0.0, x)
            y = ibuf[pl.ds(0, 16)]
            ibuf[pl.ds(0, 16)] = jnp.where(iota < pad, mw0, y)

        def lean_b(j, cin):
            o = pl.multiple_of(j * 16, 16)
            v = vbuf[pl.ds(o, 16)]
            ic = ibuf[pl.ds(o, 16)]
            inx = ibuf[pl.ds(o + 1, 16)]
            cum = plsc.cumsum(v)
            S = cum + cin
            mb = ic != inx
            plsc.addupdate_scatter(acc, [ic - mw0], S, mask=mb)
            plsc.addupdate_scatter(acc, [inx - mw0], -S, mask=mb)
            return cin + cum[15]

        return lax.fori_loop(0, WE // 16, lean_b, carry, unroll=8)

    carry = lax.fori_loop(0, nfull, lean_w, jnp.float32(0.0))

    p0t = A0 + nfull * WE
    p0t = pl.multiple_of(p0t, 16)
    st = jnp.minimum(p0t, n_atoms - W)
    st = pl.multiple_of(st, 16)
    pltpu.sync_copy(vals_hbm.at[pl.ds(st, W)], vbuf.at[pl.ds(0, W)])
    pltpu.sync_copy(idx_hbm.at[pl.ds(st, W)], ibuf.at[pl.ds(0, W)])
    base = p0t - st
    nblk = (a1 - p0t + 15) // 16

    def tail_b(j, cin):
        o = base + j * 16
        pos0 = p0t + j * 16
        v_raw = vbuf[pl.ds(o, 16)]
        ic = ibuf[pl.ds(o, 16)]
        inx = ibuf[pl.ds(o + 1, 16)]
        posv = pos0 + iota
        valid = (posv >= a0) & (posv < a1)
        v = jnp.where(valid, v_raw, 0.0)
        cum = plsc.cumsum(v)
        S = cum + cin
        mb = valid & ((ic != inx) | (posv == a1 - 1))
        msub = mb & (posv < a1 - 1)
        li = jnp.where(mb, ic - mw0, 0)
        ln = jnp.where(msub, inx - mw0, 0)
        plsc.addupdate_scatter(acc, [li], S, mask=mb)
        plsc.addupdate_scatter(acc, [ln], -S, mask=msub)
        return cin + cum[15]

    lax.fori_loop(0, nblk, tail_b, carry)

    @pl.when(w < NW - 1)
    def _():
        pltpu.sync_copy(acc.at[pl.ds(0, MPW)], out_hbm.at[pl.ds(mw0, MPW)])

    @pl.when(w == NW - 1)
    def _():
        pltpu.sync_copy(acc.at[pl.ds(0, MPW_LAST)],
                        out_hbm.at[pl.ds(mw0, MPW_LAST)])


@jax.jit
def kernel(per_atom_property, atomic_subsystem_indices):
    n_atoms = per_atom_property.shape[0]
    idx32 = atomic_subsystem_indices.astype(jnp.int32)
    targets = jnp.arange(NW, dtype=jnp.int32) * MPW
    bnd = jnp.searchsorted(idx32, targets, side="left").astype(jnp.int32)
    bnd = jnp.concatenate(
        [bnd, jnp.full((16,), n_atoms, dtype=jnp.int32)])

    mesh = plsc.VectorSubcoreMesh(core_axis_name="c", subcore_axis_name="s")
    fn = pl.kernel(
        functools.partial(_sc_body, n_atoms),
        mesh=mesh,
        compiler_params=pltpu.CompilerParams(needs_layout_passes=False),
        out_type=jax.ShapeDtypeStruct((NUM_MOL,), jnp.float32),
        scratch_types=[
            pltpu.VMEM((W + 16,), jnp.float32),
            pltpu.VMEM((W + 16,), jnp.int32),
            pltpu.VMEM((ACC_PAD,), jnp.float32),
            pltpu.VMEM((48,), jnp.int32),
        ],
    )
    return fn(per_atom_property, idx32, bnd)


# speedup vs baseline: 59.2323x; 1.9096x over previous
import functools

import jax
import jax.numpy as jnp
from jax import lax
from jax.experimental import pallas as pl
from jax.experimental.pallas import tpu as pltpu
from jax.experimental.pallas import tpu_sc as plsc

N_ATOMS_EXPECTED = 6400000
NUM_MOL = 100000
NW = 32
MPW = 3136
MPW_LAST = NUM_MOL - (NW - 1) * MPW
ACC_PAD = MPW
W = 16384
WE = W - 16


def _sc_body(n_atoms, vals_hbm, idx_hbm, bnd_hbm, out_hbm, vbuf, ibuf, acc,
             sbnd):
    c = lax.axis_index("c")
    s = lax.axis_index("s")
    w = s * 2 + c

    pltpu.sync_copy(bnd_hbm, sbnd)
    vb = sbnd[pl.ds(w, 16)]
    a0 = vb[0]
    a1 = vb[1]
    mw0 = w * MPW

    def zbody(i, _):
        acc[pl.ds(i * 16, 16)] = jnp.zeros((16,), jnp.float32)
        return 0

    lax.fori_loop(0, ACC_PAD // 16, zbody, 0)

    A0 = (a0 // 16) * 16
    iota = lax.iota(jnp.int32, 16)
    m_end15 = iota == 15
    m_low15 = iota < 15
    nfull = jnp.maximum(a1 - 1 - A0, 0) // WE

    def lean_w(k, _unused):
        p0 = A0 + k * WE
        p0 = pl.multiple_of(p0, 16)
        pltpu.sync_copy(vals_hbm.at[pl.ds(p0, W)], vbuf.at[pl.ds(0, W)])
        pltpu.sync_copy(idx_hbm.at[pl.ds(p0, W)], ibuf.at[pl.ds(0, W)])

        @pl.when(k == 0)
        def _():
            pad = a0 - A0
            x = vbuf[pl.ds(0, 16)]
            vbuf[pl.ds(0, 16)] = jnp.where(iota < pad, 0.0, x)
            y = ibuf[pl.ds(0, 16)]
            ibuf[pl.ds(0, 16)] = jnp.where(iota < pad, mw0, y)

        @plsc.parallel_loop(0, WE // 16, unroll=8)
        def _(j):
            o = pl.multiple_of(j * 16, 16)
            v = vbuf[pl.ds(o, 16)]
            ic = ibuf[pl.ds(o, 16)]
            inx = ibuf[pl.ds(o + 1, 16)]
            cum = plsc.cumsum(v)
            chg = ic != inx
            mend = chg | m_end15
            msub = chg & m_low15
            plsc.addupdate_scatter(acc, [ic - mw0], cum, mask=mend)
            plsc.addupdate_scatter(acc, [inx - mw0], -cum, mask=msub)

        return 0

    lax.fori_loop(0, nfull, lean_w, 0)

    p0t = A0 + nfull * WE
    p0t = pl.multiple_of(p0t, 16)
    st = jnp.minimum(p0t, n_atoms - W)
    st = pl.multiple_of(st, 16)
    pltpu.sync_copy(vals_hbm.at[pl.ds(st, W)], vbuf.at[pl.ds(0, W)])
    pltpu.sync_copy(idx_hbm.at[pl.ds(st, W)], ibuf.at[pl.ds(0, W)])
    base = p0t - st
    nblk = (a1 - p0t + 15) // 16

    def tail_b(j, cc):
        o = base + j * 16
        pos0 = p0t + j * 16
        v_raw = vbuf[pl.ds(o, 16)]
        ic = ibuf[pl.ds(o, 16)]
        inx = ibuf[pl.ds(o + 1, 16)]
        posv = pos0 + iota
        valid = (posv >= a0) & (posv < a1)
        v = jnp.where(valid, v_raw, 0.0)
        cum = plsc.cumsum(v)
        mend = valid & ((ic != inx) | m_end15 | (posv == a1 - 1))
        msub = valid & (ic != inx) & m_low15 & (posv < a1 - 1)
        li = jnp.where(mend, ic - mw0, 0)
        ln = jnp.where(msub, inx - mw0, 0)
        plsc.addupdate_scatter(acc, [li], cum, mask=mend)
        plsc.addupdate_scatter(acc, [ln], -cum, mask=msub)
        return cc

    lax.fori_loop(0, nblk, tail_b, 0)

    @pl.when(w < NW - 1)
    def _():
        pltpu.sync_copy(acc.at[pl.ds(0, MPW)], out_hbm.at[pl.ds(mw0, MPW)])

    @pl.when(w == NW - 1)
    def _():
        pltpu.sync_copy(acc.at[pl.ds(0, MPW_LAST)],
                        out_hbm.at[pl.ds(mw0, MPW_LAST)])


@jax.jit
def kernel(per_atom_property, atomic_subsystem_indices):
    n_atoms = per_atom_property.shape[0]
    idx32 = atomic_subsystem_indices.astype(jnp.int32)
    targets = jnp.arange(NW, dtype=jnp.int32) * MPW
    bnd = jnp.searchsorted(idx32, targets, side="left").astype(jnp.int32)
    bnd = jnp.concatenate(
        [bnd, jnp.full((16,), n_atoms, dtype=jnp.int32)])

    mesh = plsc.VectorSubcoreMesh(core_axis_name="c", subcore_axis_name="s")
    fn = pl.kernel(
        functools.partial(_sc_body, n_atoms),
        mesh=mesh,
        compiler_params=pltpu.CompilerParams(needs_layout_passes=False),
        out_type=jax.ShapeDtypeStruct((NUM_MOL,), jnp.float32),
        scratch_types=[
            pltpu.VMEM((W + 16,), jnp.float32),
            pltpu.VMEM((W + 16,), jnp.int32),
            pltpu.VMEM((ACC_PAD,), jnp.float32),
            pltpu.VMEM((48,), jnp.int32),
        ],
    )
    return fn(per_atom_property, idx32, bnd)


# speedup vs baseline: 78.6765x; 1.3283x over previous
import functools

import jax
import jax.numpy as jnp
from jax import lax
from jax.experimental import pallas as pl
from jax.experimental.pallas import tpu as pltpu
from jax.experimental.pallas import tpu_sc as plsc

NUM_MOL = 100000
NW = 32
MPW = 3136
MPW_LAST = NUM_MOL - (NW - 1) * MPW
ACC_PAD = MPW
W = 16384
WE = W - 16


def _sc_body(n_atoms, vals_hbm, idx_hbm, bnd_hbm, out_hbm,
             vbufa, ibufa, vbufb, ibufb, acc, sbnd, sema, semb):
    c = lax.axis_index("c")
    s = lax.axis_index("s")
    w = s * 2 + c

    pltpu.sync_copy(bnd_hbm, sbnd)
    vb = sbnd[pl.ds(w, 16)]
    a0 = vb[0]
    a1 = vb[1]
    mw0 = w * MPW

    def zbody(i, _):
        acc[pl.ds(i * 16, 16)] = jnp.zeros((16,), jnp.float32)
        return 0

    lax.fori_loop(0, ACC_PAD // 16, zbody, 0)

    A0 = (a0 // 16) * 16
    iota = lax.iota(jnp.int32, 16)
    m_end15 = iota == 15
    m_low15 = iota < 15
    nfull = jnp.maximum(a1 - 1 - A0, 0) // WE

    def win_start(kk):
        p0 = A0 + kk * WE
        st = jnp.minimum(p0, n_atoms - W)
        return pl.multiple_of(st, 16)

    def issue(kk, vbuf, ibuf, sem):
        st = win_start(kk)
        pltpu.async_copy(vals_hbm.at[pl.ds(st, W)], vbuf.at[pl.ds(0, W)], sem)
        pltpu.async_copy(idx_hbm.at[pl.ds(st, W)], ibuf.at[pl.ds(0, W)], sem)

    def wait(vbuf, ibuf, sem):
        pltpu.make_async_copy(vals_hbm.at[pl.ds(0, W)],
                              vbuf.at[pl.ds(0, W)], sem).wait()
        pltpu.make_async_copy(idx_hbm.at[pl.ds(0, W)],
                              ibuf.at[pl.ds(0, W)], sem).wait()

    issue(0, vbufa, ibufa, sema)

    def front_pad_fix(vbuf, ibuf):
        pad = a0 - A0
        x = vbuf[pl.ds(0, 16)]
        vbuf[pl.ds(0, 16)] = jnp.where(iota < pad, 0.0, x)
        y = ibuf[pl.ds(0, 16)]
        ibuf[pl.ds(0, 16)] = jnp.where(iota < pad, mw0, y)

    def lean_loop(vbuf, ibuf):
        @plsc.parallel_loop(0, WE // 16, unroll=8)
        def _(j):
            o = pl.multiple_of(j * 16, 16)
            v = vbuf[pl.ds(o, 16)]
            ic = ibuf[pl.ds(o, 16)]
            inx = ibuf[pl.ds(o + 1, 16)]
            cum = plsc.cumsum(v)
            chg = ic != inx
            mend = chg | m_end15
            msub = chg & m_low15
            plsc.addupdate_scatter(acc, [ic - mw0], cum, mask=mend)
            plsc.addupdate_scatter(acc, [inx - mw0], -cum, mask=msub)

    def lean_w(k, _unused):
        @pl.when(k % 2 == 0)
        def _():
            wait(vbufa, ibufa, sema)
            issue(k + 1, vbufb, ibufb, semb)

            @pl.when(k == 0)
            def _():
                front_pad_fix(vbufa, ibufa)

            lean_loop(vbufa, ibufa)

        @pl.when(k % 2 == 1)
        def _():
            wait(vbufb, ibufb, semb)
            issue(k + 1, vbufa, ibufa, sema)
            lean_loop(vbufb, ibufb)

        return 0

    lax.fori_loop(0, nfull, lean_w, 0)

    p0t = A0 + nfull * WE
    p0t = pl.multiple_of(p0t, 16)
    stt = win_start(nfull)
    base = p0t - stt
    nblk = (a1 - p0t + 15) // 16

    def tail_loop(vbuf, ibuf):
        def tail_b(j, cc):
            o = base + j * 16
            pos0 = p0t + j * 16
            v_raw = vbuf[pl.ds(o, 16)]
            ic = ibuf[pl.ds(o, 16)]
            inx = ibuf[pl.ds(o + 1, 16)]
            posv = pos0 + iota
            valid = (posv >= a0) & (posv < a1)
            v = jnp.where(valid, v_raw, 0.0)
            cum = plsc.cumsum(v)
            mend = valid & ((ic != inx) | m_end15 | (posv == a1 - 1))
            msub = valid & (ic != inx) & m_low15 & (posv < a1 - 1)
            li = jnp.where(mend, ic - mw0, 0)
            ln = jnp.where(msub, inx - mw0, 0)
            plsc.addupdate_scatter(acc, [li], cum, mask=mend)
            plsc.addupdate_scatter(acc, [ln], -cum, mask=msub)
            return cc

        lax.fori_loop(0, nblk, tail_b, 0)

    @pl.when(nfull % 2 == 0)
    def _():
        wait(vbufa, ibufa, sema)
        tail_loop(vbufa, ibufa)

    @pl.when(nfull % 2 == 1)
    def _():
        wait(vbufb, ibufb, semb)
        tail_loop(vbufb, ibufb)

    @pl.when(w < NW - 1)
    def _():
        pltpu.sync_copy(acc.at[pl.ds(0, MPW)], out_hbm.at[pl.ds(mw0, MPW)])

    @pl.when(w == NW - 1)
    def _():
        pltpu.sync_copy(acc.at[pl.ds(0, MPW_LAST)],
                        out_hbm.at[pl.ds(mw0, MPW_LAST)])


@jax.jit
def kernel(per_atom_property, atomic_subsystem_indices):
    n_atoms = per_atom_property.shape[0]
    idx32 = atomic_subsystem_indices.astype(jnp.int32)
    targets = jnp.arange(NW, dtype=jnp.int32) * MPW
    bnd = jnp.searchsorted(idx32, targets, side="left").astype(jnp.int32)
    bnd = jnp.concatenate(
        [bnd, jnp.full((16,), n_atoms, dtype=jnp.int32)])

    mesh = plsc.VectorSubcoreMesh(core_axis_name="c", subcore_axis_name="s")
    fn = pl.kernel(
        functools.partial(_sc_body, n_atoms),
        mesh=mesh,
        compiler_params=pltpu.CompilerParams(needs_layout_passes=False),
        out_type=jax.ShapeDtypeStruct((NUM_MOL,), jnp.float32),
        scratch_types=[
            pltpu.VMEM((W + 16,), jnp.float32),
            pltpu.VMEM((W + 16,), jnp.int32),
            pltpu.VMEM((W + 16,), jnp.float32),
            pltpu.VMEM((W + 16,), jnp.int32),
            pltpu.VMEM((ACC_PAD,), jnp.float32),
            pltpu.VMEM((48,), jnp.int32),
            pltpu.SemaphoreType.DMA,
            pltpu.SemaphoreType.DMA,
        ],
    )
    return fn(per_atom_property, idx32, bnd)


# speedup vs baseline: 79.0681x; 1.0050x over previous
import functools

import jax
import jax.numpy as jnp
from jax import lax
from jax.experimental import pallas as pl
from jax.experimental.pallas import tpu as pltpu
from jax.experimental.pallas import tpu_sc as plsc

NUM_MOL = 100000
NW = 32
MPW = 3136
MPW_LAST = NUM_MOL - (NW - 1) * MPW
ACC_PAD = MPW
W = 16384
WE = W - 16


def _sc_body(n_atoms, vals_hbm, idx_hbm, bnd_hbm, out_hbm,
             vbufa, ibufa, vbufb, ibufb, acc, sbnd, sema, semb):
    c = lax.axis_index("c")
    s = lax.axis_index("s")
    w = s * 2 + c

    pltpu.sync_copy(bnd_hbm, sbnd)
    vb = sbnd[pl.ds(w, 16)]
    a0 = vb[0]
    a1 = vb[1]
    mw0 = w * MPW

    def zbody(i, _):
        acc[pl.ds(i * 16, 16)] = jnp.zeros((16,), jnp.float32)
        return 0

    lax.fori_loop(0, ACC_PAD // 16, zbody, 0)

    A0 = (a0 // 16) * 16
    iota = lax.iota(jnp.int32, 16)
    m_end15 = iota == 15
    m_low15 = iota < 15
    nfull = jnp.maximum(a1 - 1 - A0, 0) // WE

    def win_start(kk):
        p0 = A0 + kk * WE
        st = jnp.minimum(p0, n_atoms - W)
        return pl.multiple_of(st, 16)

    def issue(kk, vbuf, ibuf, sem):
        st = win_start(kk)
        pltpu.async_copy(vals_hbm.at[pl.ds(st, W)], vbuf.at[pl.ds(0, W)], sem)
        pltpu.async_copy(idx_hbm.at[pl.ds(st, W)], ibuf.at[pl.ds(0, W)], sem)

    def wait(vbuf, ibuf, sem):
        pltpu.make_async_copy(vals_hbm.at[pl.ds(0, W)],
                              vbuf.at[pl.ds(0, W)], sem).wait()
        pltpu.make_async_copy(idx_hbm.at[pl.ds(0, W)],
                              ibuf.at[pl.ds(0, W)], sem).wait()

    issue(0, vbufa, ibufa, sema)

    def front_pad_fix(vbuf, ibuf):
        pad = a0 - A0
        x = vbuf[pl.ds(0, 16)]
        vbuf[pl.ds(0, 16)] = jnp.where(iota < pad, 0.0, x)
        y = ibuf[pl.ds(0, 16)]
        ibuf[pl.ds(0, 16)] = jnp.where(iota < pad, mw0, y)

    def lean_loop(vbuf, ibuf):
        @plsc.parallel_loop(0, WE // 16, unroll=8)
        def _(j):
            o = pl.multiple_of(j * 16, 16)
            v = vbuf[pl.ds(o, 16)]
            ic = ibuf[pl.ds(o, 16)]
            inx = ibuf[pl.ds(o + 1, 16)]
            cum = plsc.cumsum(v)
            chg = ic != inx
            mend = chg | m_end15
            msub = chg & m_low15
            plsc.addupdate_scatter(acc, [ic - mw0], cum, mask=mend)
            plsc.addupdate_scatter(acc, [inx - mw0], -cum, mask=msub)

    def lean_w(k, _unused):
        @pl.when(k % 2 == 0)
        def _():
            wait(vbufa, ibufa, sema)
            issue(k + 1, vbufb, ibufb, semb)

            @pl.when(k == 0)
            def _():
                front_pad_fix(vbufa, ibufa)

            lean_loop(vbufa, ibufa)

        @pl.when(k % 2 == 1)
        def _():
            wait(vbufb, ibufb, semb)
            issue(k + 1, vbufa, ibufa, sema)
            lean_loop(vbufb, ibufb)

        return 0

    lax.fori_loop(0, nfull, lean_w, 0)

    p0t = A0 + nfull * WE
    p0t = pl.multiple_of(p0t, 16)
    stt = win_start(nfull)
    base = p0t - stt
    nblk = (a1 - p0t + 15) // 16

    def tail_loop(vbuf, ibuf):
        def tail_b(j, cc):
            o = base + j * 16
            pos0 = p0t + j * 16
            v_raw = vbuf[pl.ds(o, 16)]
            ic = ibuf[pl.ds(o, 16)]
            inx = ibuf[pl.ds(o + 1, 16)]
            posv = pos0 + iota
            valid = (posv >= a0) & (posv < a1)
            v = jnp.where(valid, v_raw, 0.0)
            cum = plsc.cumsum(v)
            mend = valid & ((ic != inx) | m_end15 | (posv == a1 - 1))
            msub = valid & (ic != inx) & m_low15 & (posv < a1 - 1)
            li = jnp.where(mend, ic - mw0, 0)
            ln = jnp.where(msub, inx - mw0, 0)
            plsc.addupdate_scatter(acc, [li], cum, mask=mend)
            plsc.addupdate_scatter(acc, [ln], -cum, mask=msub)
            return cc

        lax.fori_loop(0, nblk, tail_b, 0)

    @pl.when(nfull % 2 == 0)
    def _():
        wait(vbufa, ibufa, sema)
        tail_loop(vbufa, ibufa)

    @pl.when(nfull % 2 == 1)
    def _():
        wait(vbufb, ibufb, semb)
        tail_loop(vbufb, ibufb)

    @pl.when(w < NW - 1)
    def _():
        pltpu.sync_copy(acc.at[pl.ds(0, MPW)], out_hbm.at[pl.ds(mw0, MPW)])

    @pl.when(w == NW - 1)
    def _():
        pltpu.sync_copy(acc.at[pl.ds(0, MPW_LAST)],
                        out_hbm.at[pl.ds(mw0, MPW_LAST)])


@jax.jit
def kernel(per_atom_property, atomic_subsystem_indices):
    n_atoms = per_atom_property.shape[0]
    idx32 = atomic_subsystem_indices.astype(jnp.int32)
    targets = jnp.arange(NW, dtype=jnp.int32) * MPW
    lo = jnp.zeros((NW,), jnp.int32)
    hi = jnp.full((NW,), n_atoms, dtype=jnp.int32)
    for _ in range(max(1, (n_atoms).bit_length())):
        upd = lo < hi
        mid = (lo + hi) >> 1
        v = idx32[jnp.minimum(mid, n_atoms - 1)]
        go = upd & (v < targets)
        lo = jnp.where(go, mid + 1, lo)
        hi = jnp.where(upd & jnp.logical_not(go), mid, hi)
    bnd = jnp.concatenate(
        [lo, jnp.full((16,), n_atoms, dtype=jnp.int32)])

    mesh = plsc.VectorSubcoreMesh(core_axis_name="c", subcore_axis_name="s")
    fn = pl.kernel(
        functools.partial(_sc_body, n_atoms),
        mesh=mesh,
        compiler_params=pltpu.CompilerParams(needs_layout_passes=False),
        out_type=jax.ShapeDtypeStruct((NUM_MOL,), jnp.float32),
        scratch_types=[
            pltpu.VMEM((W + 16,), jnp.float32),
            pltpu.VMEM((W + 16,), jnp.int32),
            pltpu.VMEM((W + 16,), jnp.float32),
            pltpu.VMEM((W + 16,), jnp.int32),
            pltpu.VMEM((ACC_PAD,), jnp.float32),
            pltpu.VMEM((48,), jnp.int32),
            pltpu.SemaphoreType.DMA,
            pltpu.SemaphoreType.DMA,
        ],
    )
    return fn(per_atom_property, idx32, bnd)


# speedup vs baseline: 106.4471x; 1.3463x over previous
import functools

import jax
import jax.numpy as jnp
from jax import lax
from jax.experimental import pallas as pl
from jax.experimental.pallas import tpu as pltpu
from jax.experimental.pallas import tpu_sc as plsc

NUM_MOL = 100000
NW = 32
MPW = 3136
MPW_LAST = NUM_MOL - (NW - 1) * MPW
ACC_PAD = MPW
W = 16384
WE = W - 16
CS = 2048


def _sc_body(n_atoms, vals_hbm, idx_hbm, bnd_hbm, out_hbm,
             vbufa, ibufa, vbufb, ibufb, acc, sbnd, sema, semb):
    c = lax.axis_index("c")
    s = lax.axis_index("s")
    w = s * 2 + c

    pltpu.sync_copy(bnd_hbm, sbnd)
    a0 = sbnd[pl.ds(w, 16)][0]
    a1 = sbnd[pl.ds(w + NW, 16)][0]
    mw0 = w * MPW
    mpw_w = jnp.minimum(mw0 + MPW, NUM_MOL) - mw0

    sent = jnp.full((16,), NUM_MOL, dtype=jnp.int32)
    ibufa[pl.ds(W, 16)] = sent
    ibufb[pl.ds(W, 16)] = sent

    def zbody(i, _):
        acc[pl.ds(i * 16, 16)] = jnp.zeros((16,), jnp.float32)
        return 0

    lax.fori_loop(0, ACC_PAD // 16, zbody, 0)

    iota = lax.iota(jnp.int32, 16)
    m_end15 = iota == 15
    m_low15 = iota < 15
    mpw_u = jnp.full((16,), mpw_w, dtype=jnp.int32)

    nwin = jnp.maximum((a1 - a0 + WE - 1) // WE, 1)
    nfull = nwin - 1

    def issue(kk, vbuf, ibuf, sem):
        p0 = a0 + kk * WE
        st = jnp.minimum(p0, n_atoms - W)
        st = pl.multiple_of(st, 16)
        pltpu.async_copy(vals_hbm.at[pl.ds(st, W)], vbuf.at[pl.ds(0, W)], sem)
        pltpu.async_copy(idx_hbm.at[pl.ds(st, W)], ibuf.at[pl.ds(0, W)], sem)

    def wait(vbuf, ibuf, sem):
        pltpu.make_async_copy(vals_hbm.at[pl.ds(0, W)],
                              vbuf.at[pl.ds(0, W)], sem).wait()
        pltpu.make_async_copy(idx_hbm.at[pl.ds(0, W)],
                              ibuf.at[pl.ds(0, W)], sem).wait()

    issue(0, vbufa, ibufa, sema)

    def body_at(vbuf, ibuf, base, j):
        o = base + j * 16
        v = vbuf[pl.ds(o, 16)]
        ic = ibuf[pl.ds(o, 16)]
        inx = ibuf[pl.ds(o + 1, 16)]
        cum = plsc.cumsum(v)
        li = ic - mw0
        ln = inx - mw0
        chg = ic != inx
        in_i = plsc.bitcast(li, jnp.uint32) < plsc.bitcast(mpw_u, jnp.uint32)
        in_n = plsc.bitcast(ln, jnp.uint32) < plsc.bitcast(mpw_u, jnp.uint32)
        mend = (chg | m_end15) & in_i
        msub = chg & m_low15 & in_n
        plsc.addupdate_scatter(acc, [li], cum, mask=mend)
        plsc.addupdate_scatter(acc, [ln], -cum, mask=msub)

    def lean_loop(vbuf, ibuf):
        @plsc.parallel_loop(0, WE // 16, unroll=8)
        def _(j):
            body_at(vbuf, ibuf, 0, j)

    def lean_w(k, _unused):
        @pl.when(k % 2 == 0)
        def _():
            wait(vbufa, ibufa, sema)
            issue(k + 1, vbufb, ibufb, semb)
            lean_loop(vbufa, ibufa)

        @pl.when(k % 2 == 1)
        def _():
            wait(vbufb, ibufb, semb)
            issue(k + 1, vbufa, ibufa, sema)
            lean_loop(vbufb, ibufb)

        return 0

    lax.fori_loop(0, nfull, lean_w, 0)

    p0t = a0 + nfull * WE
    stt = jnp.minimum(p0t, n_atoms - W)
    base = pl.multiple_of(p0t - stt, 16)
    nblk = (a1 - p0t + 15) // 16

    def tail_loop(vbuf, ibuf):
        @plsc.parallel_loop(0, nblk, unroll=4)
        def _(j):
            body_at(vbuf, ibuf, base, j)

    @pl.when(nfull % 2 == 0)
    def _():
        wait(vbufa, ibufa, sema)
        tail_loop(vbufa, ibufa)

    @pl.when(nfull % 2 == 1)
    def _():
        wait(vbufb, ibufb, semb)
        tail_loop(vbufb, ibufb)

    @pl.when(w < NW - 1)
    def _():
        pltpu.sync_copy(acc.at[pl.ds(0, MPW)], out_hbm.at[pl.ds(mw0, MPW)])

    @pl.when(w == NW - 1)
    def _():
        pltpu.sync_copy(acc.at[pl.ds(0, MPW_LAST)],
                        out_hbm.at[pl.ds(mw0, MPW_LAST)])


@jax.jit
def kernel(per_atom_property, atomic_subsystem_indices):
    n_atoms = per_atom_property.shape[0]
    idx32 = atomic_subsystem_indices.astype(jnp.int32)
    cs = idx32[::CS]
    t0 = jnp.arange(NW, dtype=jnp.int32) * MPW
    t1 = jnp.minimum(t0 + MPW, NUM_MOL)
    j0 = jnp.sum(cs[None, :] < t0[:, None], axis=1, dtype=jnp.int32)
    j1 = jnp.sum(cs[None, :] < t1[:, None], axis=1, dtype=jnp.int32)
    a0s = jnp.maximum(0, (j0 - 1) * CS)
    a1s = jnp.minimum(n_atoms, j1 * CS)
    bnd = jnp.concatenate(
        [a0s, a1s, jnp.full((16,), n_atoms, dtype=jnp.int32)])

    mesh = plsc.VectorSubcoreMesh(core_axis_name="c", subcore_axis_name="s")
    fn = pl.kernel(
        functools.partial(_sc_body, n_atoms),
        mesh=mesh,
        compiler_params=pltpu.CompilerParams(needs_layout_passes=False),
        out_type=jax.ShapeDtypeStruct((NUM_MOL,), jnp.float32),
        scratch_types=[
            pltpu.VMEM((W + 16,), jnp.float32),
            pltpu.VMEM((W + 16,), jnp.int32),
            pltpu.VMEM((W + 16,), jnp.float32),
            pltpu.VMEM((W + 16,), jnp.int32),
            pltpu.VMEM((ACC_PAD,), jnp.float32),
            pltpu.VMEM((80,), jnp.int32),
            pltpu.SemaphoreType.DMA,
            pltpu.SemaphoreType.DMA,
        ],
    )
    return fn(per_atom_property, idx32, bnd)


# speedup vs baseline: 109.1296x; 1.0252x over previous
import functools

import jax
import jax.numpy as jnp
from jax import lax
from jax.experimental import pallas as pl
from jax.experimental.pallas import tpu as pltpu
from jax.experimental.pallas import tpu_sc as plsc

NUM_MOL = 100000
NW = 32
MPW = 3136
MPW_LAST = NUM_MOL - (NW - 1) * MPW
ACC_PAD = MPW
W = 16384
WE = W - 16
CS = 2048


def _sc_body(n_atoms, vals_hbm, idx_hbm, out_hbm,
             vbufa, ibufa, vbufb, ibufb, acc, cilist, csbuf, sema, semb):
    c = lax.axis_index("c")
    s = lax.axis_index("s")
    w = s * 2 + c
    mw0 = w * MPW
    mpw_w = jnp.minimum(mw0 + MPW, NUM_MOL) - mw0

    iota0 = lax.iota(jnp.int32, 16)
    ncs = n_atoms // CS
    ncs_pad = ((ncs + 15) // 16) * 16

    off = w * 97

    def bld(i, _):
        slot = i * 16 + iota0
        pos = slot + off
        pos = jnp.where(pos >= ncs, pos - ncs, pos)
        pos = jnp.where(slot >= ncs, 0, pos)
        cilist[pl.ds(i * 16, 16)] = pos * CS
        return 0

    lax.fori_loop(0, ncs_pad // 16, bld, 0)
    pltpu.async_copy(idx_hbm.at[cilist], csbuf, sema).wait()

    t0v = jnp.full((16,), mw0, dtype=jnp.int32)
    t1v = jnp.full((16,), mw0 + mpw_w, dtype=jnp.int32)

    def cnt(i, carry):
        c0, c1 = carry
        slot = i * 16 + iota0
        valid = slot < ncs
        cs_v = csbuf[pl.ds(i * 16, 16)]
        c0 = c0 + plsc.all_reduce_population_count((cs_v < t0v) & valid)
        c1 = c1 + plsc.all_reduce_population_count((cs_v < t1v) & valid)
        return c0, c1

    zc = jnp.zeros((16,), jnp.int32)
    j0v, j1v = lax.fori_loop(0, ncs_pad // 16, cnt, (zc, zc))
    a0 = jnp.maximum(0, (j0v[0] - 1) * CS)
    a1 = jnp.minimum(n_atoms, j1v[0] * CS)

    sent = jnp.full((16,), NUM_MOL, dtype=jnp.int32)
    ibufa[pl.ds(W, 16)] = sent
    ibufb[pl.ds(W, 16)] = sent

    def zbody(i, _):
        acc[pl.ds(i * 16, 16)] = jnp.zeros((16,), jnp.float32)
        return 0

    lax.fori_loop(0, ACC_PAD // 16, zbody, 0)

    iota = lax.iota(jnp.int32, 16)
    m_end15 = iota == 15
    m_low15 = iota < 15
    mpw_u = jnp.full((16,), mpw_w, dtype=jnp.int32)

    nwin = jnp.maximum((a1 - a0 + WE - 1) // WE, 1)
    nfull = nwin - 1

    def issue(kk, vbuf, ibuf, sem):
        p0 = a0 + kk * WE
        st = jnp.minimum(p0, n_atoms - W)
        st = pl.multiple_of(st, 16)
        pltpu.async_copy(vals_hbm.at[pl.ds(st, W)], vbuf.at[pl.ds(0, W)], sem)
        pltpu.async_copy(idx_hbm.at[pl.ds(st, W)], ibuf.at[pl.ds(0, W)], sem)

    def wait(vbuf, ibuf, sem):
        pltpu.make_async_copy(vals_hbm.at[pl.ds(0, W)],
                              vbuf.at[pl.ds(0, W)], sem).wait()
        pltpu.make_async_copy(idx_hbm.at[pl.ds(0, W)],
                              ibuf.at[pl.ds(0, W)], sem).wait()

    issue(0, vbufa, ibufa, sema)

    def body_at(vbuf, ibuf, base, j):
        o = base + j * 16
        v = vbuf[pl.ds(o, 16)]
        ic = ibuf[pl.ds(o, 16)]
        inx = ibuf[pl.ds(o + 1, 16)]
        cum = plsc.cumsum(v)
        li = ic - mw0
        ln = inx - mw0
        chg = ic != inx
        in_i = plsc.bitcast(li, jnp.uint32) < plsc.bitcast(mpw_u, jnp.uint32)
        in_n = plsc.bitcast(ln, jnp.uint32) < plsc.bitcast(mpw_u, jnp.uint32)
        mend = (chg | m_end15) & in_i
        msub = chg & m_low15 & in_n
        plsc.addupdate_scatter(acc, [li], cum, mask=mend)
        plsc.addupdate_scatter(acc, [ln], -cum, mask=msub)

    def lean_loop(vbuf, ibuf):
        @plsc.parallel_loop(0, WE // 16, unroll=8)
        def _(j):
            body_at(vbuf, ibuf, 0, j)

    def lean_w(k, _unused):
        @pl.when(k % 2 == 0)
        def _():
            wait(vbufa, ibufa, sema)
            issue(k + 1, vbufb, ibufb, semb)
            lean_loop(vbufa, ibufa)

        @pl.when(k % 2 == 1)
        def _():
            wait(vbufb, ibufb, semb)
            issue(k + 1, vbufa, ibufa, sema)
            lean_loop(vbufb, ibufb)

        return 0

    lax.fori_loop(0, nfull, lean_w, 0)

    p0t = a0 + nfull * WE
    stt = jnp.minimum(p0t, n_atoms - W)
    base = pl.multiple_of(p0t - stt, 16)
    nblk = (a1 - p0t + 15) // 16

    def tail_loop(vbuf, ibuf):
        @plsc.parallel_loop(0, nblk, unroll=4)
        def _(j):
            body_at(vbuf, ibuf, base, j)

    @pl.when(nfull % 2 == 0)
    def _():
        wait(vbufa, ibufa, sema)
        tail_loop(vbufa, ibufa)

    @pl.when(nfull % 2 == 1)
    def _():
        wait(vbufb, ibufb, semb)
        tail_loop(vbufb, ibufb)

    @pl.when(w < NW - 1)
    def _():
        pltpu.sync_copy(acc.at[pl.ds(0, MPW)], out_hbm.at[pl.ds(mw0, MPW)])

    @pl.when(w == NW - 1)
    def _():
        pltpu.sync_copy(acc.at[pl.ds(0, MPW_LAST)],
                        out_hbm.at[pl.ds(mw0, MPW_LAST)])


@jax.jit
def kernel(per_atom_property, atomic_subsystem_indices):
    n_atoms = per_atom_property.shape[0]
    idx32 = atomic_subsystem_indices.astype(jnp.int32)
    ncs_pad = ((n_atoms // CS + 15) // 16) * 16

    mesh = plsc.VectorSubcoreMesh(core_axis_name="c", subcore_axis_name="s")
    fn = pl.kernel(
        functools.partial(_sc_body, n_atoms),
        mesh=mesh,
        compiler_params=pltpu.CompilerParams(needs_layout_passes=False),
        out_type=jax.ShapeDtypeStruct((NUM_MOL,), jnp.float32),
        scratch_types=[
            pltpu.VMEM((W + 16,), jnp.float32),
            pltpu.VMEM((W + 16,), jnp.int32),
            pltpu.VMEM((W + 16,), jnp.float32),
            pltpu.VMEM((W + 16,), jnp.int32),
            pltpu.VMEM((ACC_PAD,), jnp.float32),
            pltpu.VMEM((ncs_pad,), jnp.int32),
            pltpu.VMEM((ncs_pad,), jnp.int32),
            pltpu.SemaphoreType.DMA,
            pltpu.SemaphoreType.DMA,
        ],
    )
    return fn(per_atom_property, idx32)
